# baseline (speedup 1.0000x reference)
import jax
import jax.numpy as jnp
from jax import lax
from jax.experimental import pallas as pl
from jax.experimental.pallas import tpu as pltpu

N = 100000
HIDDEN = 128
ATTN = 128
B = 512
SUB = 1024
NSUBBLK = 8
BLK = SUB * NSUBBLK
NBLK = (N + BLK - 1) // BLK
NSUB = NBLK * NSUBBLK
SW = 16

NEG = -1e30
TAILV = N - (NBLK - 1) * BLK


def _body(base_ref, width_ref, x_ref, b3_ref, w1_ref, b1_ref, w2_ref,
          out_ref, d_scr):
    pid = pl.program_id(0)

    @pl.when(pid == 0)
    def _():
        d_scr[...] = jnp.zeros((B, 1), jnp.float32)
        out_ref[...] = jnp.zeros((B, HIDDEN), jnp.float32)

    @pl.when(pid == NBLK - 1)
    def _():
        x_ref[TAILV:, :] = jnp.zeros((BLK - TAILV, HIDDEN), jnp.float32)

    w1 = w1_ref[...].astype(jnp.bfloat16)
    w2f = w2_ref[...]
    w2 = w2f.astype(jnp.bfloat16)
    b1 = b1_ref[...]
    m0 = jnp.sum(jnp.abs(w2f))

    for j in range(NSUBBLK):
        xb = x_ref[j * SUB:(j + 1) * SUB, :].astype(jnp.bfloat16)
        h = jnp.tanh(
            lax.dot_general(xb, w1, (((1,), (0,)), ((), ())),
                            preferred_element_type=jnp.float32)
            + b1
        )
        s_row = lax.dot_general(w2, h.astype(jnp.bfloat16),
                                (((1,), (1,)), ((), ())),
                                preferred_element_type=jnp.float32)

        gid = pid * BLK + j * SUB + lax.broadcasted_iota(jnp.int32, (1, SUB), 1)
        valid = gid < N

        b_row = b3_ref[0, j, :].reshape(1, SUB)
        e = jnp.where(valid, jnp.exp(s_row - m0), 0.0)
        sp = pid * NSUBBLK + j
        base = base_ref[sp]
        width = width_ref[sp]

        def accumulate(seg_col, d_ref_sl, o_ref_sl, a_e=e, a_x=xb):
            a = jnp.where(seg_col == b_row, a_e, 0.0)
            d_ref_sl[...] += jnp.sum(a, axis=1, keepdims=True)
            o_ref_sl[...] += lax.dot_general(
                a.astype(jnp.bfloat16), a_x, (((1,), (0,)), ((), ())),
                preferred_element_type=jnp.float32)

        @pl.when(width <= SW)
        def _():
            cbase = jnp.minimum(base, B - SW)
            seg_col = cbase + lax.broadcasted_iota(jnp.int32, (SW, 1), 0)
            accumulate(seg_col, d_scr.at[pl.ds(cbase, SW), :],
                       out_ref.at[pl.ds(cbase, SW), :])

        @pl.when(width > SW)
        def _():
            seg_col = lax.broadcasted_iota(jnp.int32, (B, 1), 0)
            accumulate(seg_col, d_scr.at[...], out_ref.at[...])

    @pl.when(pid == pl.num_programs(0) - 1)
    def _():
        out_ref[...] = out_ref[...] / (d_scr[...] + 1e-16)


def kernel(x, batch, W1, b1, W2, b2):
    del b2
    bi = batch.astype(jnp.int32)
    b3 = jnp.pad(bi, (0, NBLK * BLK - N), constant_values=B - 1).reshape(
        NBLK, NSUBBLK, SUB)
    idx = jnp.arange(NSUB, dtype=jnp.int32)
    bases = bi[jnp.minimum(idx * SUB, N - 1)]
    lasts = bi[jnp.minimum((idx + 1) * SUB - 1, N - 1)]
    widths = lasts - bases + 1
    b1r = b1.reshape(1, HIDDEN)
    w2r = W2.reshape(1, ATTN)

    pooled = pl.pallas_call(
        _body,
        grid=(NBLK,),
        in_specs=[
            pl.BlockSpec(memory_space=pltpu.SMEM),
            pl.BlockSpec(memory_space=pltpu.SMEM),
            pl.BlockSpec((BLK, HIDDEN), lambda i: (i, 0)),
            pl.BlockSpec((1, NSUBBLK, SUB), lambda i: (i, 0, 0)),
            pl.BlockSpec((HIDDEN, ATTN), lambda i: (0, 0)),
            pl.BlockSpec((1, ATTN), lambda i: (0, 0)),
            pl.BlockSpec((1, ATTN), lambda i: (0, 0)),
        ],
        out_specs=pl.BlockSpec((B, HIDDEN), lambda i: (0, 0)),
        out_shape=jax.ShapeDtypeStruct((B, HIDDEN), jnp.float32),
        scratch_shapes=[
            pltpu.VMEM((B, 1), jnp.float32),
        ],
    )(bases, widths, x, b3, W1, b1r, w2r)

    return pooled


# speedup vs baseline: 1.0431x; 1.0431x over previous
import jax
import jax.numpy as jnp
from jax import lax
from jax.experimental import pallas as pl
from jax.experimental.pallas import tpu as pltpu

N = 100000
HIDDEN = 128
ATTN = 128
B = 512
SUB = 1024
NSUBBLK = 4
BLK = SUB * NSUBBLK
NBLK = (N + BLK - 1) // BLK
NSUB = NBLK * NSUBBLK
SW = 16

NEG = -1e30
TAILV = N - (NBLK - 1) * BLK


def _body(base_ref, width_ref, x_ref, b3_ref, w1_ref, b1_ref, w2_ref,
          out_ref, d_scr):
    pid = pl.program_id(0)

    @pl.when(pid == 0)
    def _():
        d_scr[...] = jnp.zeros((B, 1), jnp.float32)
        out_ref[...] = jnp.zeros((B, HIDDEN), jnp.float32)

    @pl.when(pid == NBLK - 1)
    def _():
        x_ref[TAILV:, :] = jnp.zeros((BLK - TAILV, HIDDEN), jnp.float32)

    w1 = w1_ref[...].astype(jnp.bfloat16)
    w2f = w2_ref[...]
    w2 = w2f.astype(jnp.bfloat16)
    b1 = b1_ref[...]
    m0 = jnp.sum(jnp.abs(w2f))

    for j in range(NSUBBLK):
        xb = x_ref[j * SUB:(j + 1) * SUB, :].astype(jnp.bfloat16)
        h = jnp.tanh(
            lax.dot_general(xb, w1, (((1,), (0,)), ((), ())),
                            preferred_element_type=jnp.float32)
            + b1
        )
        s_row = lax.dot_general(w2, h.astype(jnp.bfloat16),
                                (((1,), (1,)), ((), ())),
                                preferred_element_type=jnp.float32)

        gid = pid * BLK + j * SUB + lax.broadcasted_iota(jnp.int32, (1, SUB), 1)
        valid = gid < N

        b_row = b3_ref[0, j, :].reshape(1, SUB)
        e = jnp.where(valid, jnp.exp(s_row - m0), 0.0)
        sp = pid * NSUBBLK + j
        base = base_ref[sp]
        width = width_ref[sp]

        def accumulate(seg_col, d_ref_sl, o_ref_sl, a_e=e, a_x=xb):
            a = jnp.where(seg_col == b_row, a_e, 0.0)
            d_ref_sl[...] += jnp.sum(a, axis=1, keepdims=True)
            o_ref_sl[...] += lax.dot_general(
                a.astype(jnp.bfloat16), a_x, (((1,), (0,)), ((), ())),
                preferred_element_type=jnp.float32)

        @pl.when(width <= SW)
        def _():
            cbase = jnp.minimum(base, B - SW)
            seg_col = cbase + lax.broadcasted_iota(jnp.int32, (SW, 1), 0)
            accumulate(seg_col, d_scr.at[pl.ds(cbase, SW), :],
                       out_ref.at[pl.ds(cbase, SW), :])

        @pl.when(width > SW)
        def _():
            seg_col = lax.broadcasted_iota(jnp.int32, (B, 1), 0)
            accumulate(seg_col, d_scr.at[...], out_ref.at[...])

    @pl.when(pid == pl.num_programs(0) - 1)
    def _():
        out_ref[...] = out_ref[...] / (d_scr[...] + 1e-16)


def kernel(x, batch, W1, b1, W2, b2):
    del b2
    bi = batch.astype(jnp.int32)
    b3 = jnp.pad(bi, (0, NBLK * BLK - N), constant_values=B - 1).reshape(
        NBLK, NSUBBLK, SUB)
    idx = jnp.arange(NSUB, dtype=jnp.int32)
    bases = bi[jnp.minimum(idx * SUB, N - 1)]
    lasts = bi[jnp.minimum((idx + 1) * SUB - 1, N - 1)]
    widths = lasts - bases + 1
    b1r = b1.reshape(1, HIDDEN)
    w2r = W2.reshape(1, ATTN)

    pooled = pl.pallas_call(
        _body,
        grid=(NBLK,),
        in_specs=[
            pl.BlockSpec(memory_space=pltpu.SMEM),
            pl.BlockSpec(memory_space=pltpu.SMEM),
            pl.BlockSpec((BLK, HIDDEN), lambda i: (i, 0)),
            pl.BlockSpec((1, NSUBBLK, SUB), lambda i: (i, 0, 0)),
            pl.BlockSpec((HIDDEN, ATTN), lambda i: (0, 0)),
            pl.BlockSpec((1, ATTN), lambda i: (0, 0)),
            pl.BlockSpec((1, ATTN), lambda i: (0, 0)),
        ],
        out_specs=pl.BlockSpec((B, HIDDEN), lambda i: (0, 0)),
        out_shape=jax.ShapeDtypeStruct((B, HIDDEN), jnp.float32),
        scratch_shapes=[
            pltpu.VMEM((B, 1), jnp.float32),
        ],
    )(bases, widths, x, b3, W1, b1r, w2r)

    return pooled
